# baseline (speedup 1.0000x reference)
import functools

import jax
import jax.numpy as jnp
from jax import lax
from jax.experimental import pallas as pl
from jax.experimental.pallas import tpu as pltpu
from jax.experimental.pallas import tpu_sc as plsc

R, S, D = 16384, 200, 32
B = R * S
IDX_MINOR = 128
NROWS = B // IDX_MINOR
NW = 32
CHUNK = 1024
K = CHUNK // IDX_MINOR
ROWS_PW = NROWS // NW
NCHUNK = ROWS_PW // K

_mesh = plsc.VectorSubcoreMesh(core_axis_name="c", subcore_axis_name="s")


@functools.partial(
    pl.kernel,
    mesh=_mesh,
    out_type=jax.ShapeDtypeStruct((B, D), jnp.float32),
    scratch_types=[
        pltpu.VMEM((2, K, IDX_MINOR), jnp.int32),
        pltpu.VMEM((2, CHUNK, D), jnp.float32),
        pltpu.SemaphoreType.DMA((2,)),
        pltpu.SemaphoreType.DMA((2,)),
    ],
    compiler_params=pltpu.CompilerParams(use_tc_tiling_on_sc=False),
)
def _sc_gather(table_hbm, idx_hbm, out_hbm, idx_v, rows_v, gsem, ssem):
    wid = lax.axis_index("s") * 2 + lax.axis_index("c")
    row0 = wid * ROWS_PW

    def fire_chunk(c, b):
        base_row = row0 + c * K
        pltpu.sync_copy(idx_hbm.at[pl.ds(base_row, K)], idx_v.at[b])
        for j in range(K):
            pltpu.async_copy(
                table_hbm.at[idx_v.at[b].at[j]],
                rows_v.at[b].at[pl.ds(j * IDX_MINOR, IDX_MINOR)],
                gsem.at[b],
            )

    def wait_gathers(b):
        pltpu.make_async_copy(
            out_hbm.at[pl.ds(0, CHUNK)], rows_v.at[b], gsem.at[b]
        ).wait()

    def fire_store(c, b):
        base_row = row0 + c * K
        pltpu.async_copy(
            rows_v.at[b],
            out_hbm.at[pl.ds(base_row * IDX_MINOR, CHUNK)],
            ssem.at[b],
        )

    def wait_store(b):
        pltpu.make_async_copy(
            rows_v.at[b], out_hbm.at[pl.ds(0, CHUNK)], ssem.at[b]
        ).wait()

    fire_chunk(0, 0)
    fire_chunk(1, 1)
    wait_gathers(0)
    fire_store(0, 0)

    def body(g, _):
        for b in range(2):
            c = 2 + 2 * g + b
            wait_store(b)
            fire_chunk(c, b)
            wait_gathers(1 - b)
            fire_store(c - 1, 1 - b)
        return ()

    lax.fori_loop(0, (NCHUNK - 2) // 2, body, (), unroll=False)

    wait_gathers(1)
    fire_store(NCHUNK - 1, 1)
    wait_store(0)
    wait_store(1)


def kernel(visit_segments, embedding_weight):
    idx = visit_segments.reshape(NROWS, IDX_MINOR).astype(jnp.int32)
    out = _sc_gather(embedding_weight, idx)
    return out.reshape(R, S, D)
